# baseline (speedup 1.0000x reference)
import functools

import jax
import jax.numpy as jnp
from jax import lax
from jax.experimental import pallas as pl
from jax.experimental.pallas import tpu as pltpu
from jax.experimental.pallas import tpu_sc as plsc

_N = 68
_E = 544
_H = 128
_C = 8
_LANES = 16
_NSUB = 4
_NVEC = _E // _LANES
_WVEC = 9


def _sc_build_adj(edge_index):
    mesh = plsc.VectorSubcoreMesh(
        core_axis_name="c", subcore_axis_name="s",
        num_cores=1, num_subcores=_NSUB)

    @functools.partial(
        pl.kernel,
        out_type=jax.ShapeDtypeStruct((_NSUB, _N, _N), jnp.float32),
        mesh=mesh,
        scratch_types=[
            pltpu.VMEM((_E,), jnp.int32),
            pltpu.VMEM((_E,), jnp.int32),
            pltpu.VMEM((_N, _N), jnp.float32),
            pltpu.SemaphoreType.DMA,
        ],
        compiler_params=pltpu.CompilerParams(
            needs_layout_passes=False,
            disable_bounds_checks=True,
            disable_semaphore_checks=True,
            skip_device_barrier=True,
        ),
    )
    def build(ei_hbm, out_hbm, src_v, dst_v, acc_v, sem):
        sid = lax.axis_index("s")

        @pl.when(sid < _NSUB)
        def _():
            lo = sid * _NVEC // _NSUB
            hi = (sid + 1) * _NVEC // _NSUB
            cp_s = pltpu.async_copy(ei_hbm.at[0], src_v, sem)
            cp_d = pltpu.async_copy(ei_hbm.at[1], dst_v, sem)

            zeros = jnp.zeros((_LANES,), jnp.float32)

            def zero_body(i, carry):
                for r in range(4):
                    acc_v[i * 4 + r, pl.ds(0, _LANES)] = zeros
                    acc_v[i * 4 + r, pl.ds(16, _LANES)] = zeros
                    acc_v[i * 4 + r, pl.ds(32, _LANES)] = zeros
                    acc_v[i * 4 + r, pl.ds(48, _LANES)] = zeros
                    acc_v[i * 4 + r, pl.ds(_N - _LANES, _LANES)] = zeros
                return carry

            lax.fori_loop(0, _N // 4, zero_body, 0)

            cp_s.wait()
            cp_d.wait()

            ones = jnp.ones((_LANES,), jnp.float32)
            lane = lax.iota(jnp.int32, _LANES)

            def edge_body(j, carry):
                s_ids = src_v[pl.ds((lo + j) * _LANES, _LANES)]
                d_ids = dst_v[pl.ds((lo + j) * _LANES, _LANES)]
                for l in range(_LANES):
                    plsc.addupdate_scatter(
                        acc_v, [d_ids, s_ids], ones, mask=lane == l)
                return carry

            lax.fori_loop(0, hi - lo, edge_body, 0)
            pltpu.sync_copy(acc_v, out_hbm.at[sid])

    return build(edge_index)


def _tc_body(ap_ref, p_ref, s_ref, w4_ref, bb_ref, out_ref):
    def dot(p, q):
        return lax.dot_general(p, q, (((1,), (0,)), ((), ())),
                               preferred_element_type=jnp.float32)

    def dot_rt(p, q):
        return lax.dot_general(p, q, (((1,), (1,)), ((), ())),
                               preferred_element_type=jnp.float32)

    a = jnp.sum(ap_ref[...], axis=0)
    h = p_ref[0:_N, :]
    wr1 = p_ref[_N:_N + _H, :]
    wo1 = p_ref[_N + _H:, :]

    for wr, wo, b in ((wr1, wo1, bb_ref[0]),
                      (w4_ref[0], w4_ref[1], bb_ref[1]),
                      (w4_ref[2], w4_ref[3], bb_ref[2])):
        h = jnp.maximum(
            dot(a, dot_rt(h, wr)) + dot_rt(h, wo) + b[None, :], 0.0)

    st = s_ref[0]
    m = jnp.max(st, axis=1, keepdims=True)
    e = jnp.exp(st - m)
    ss = e / jnp.sum(e, axis=1, keepdims=True)
    out_ref[...] = lax.dot_general(ss, h, (((0,), (0,)), ((), ())),
                                   preferred_element_type=jnp.float32)


def kernel(x, edge_index, adj, s,
           W_rel1, b_rel1, W_root1, b_root1,
           W_rel2, b_rel2, W_root2, b_root2,
           W_rel3, b_rel3, W_root3, b_root3):
    del adj
    ap = _sc_build_adj(edge_index)

    p = jnp.concatenate([x, W_rel1, W_root1], axis=0)
    w4 = jnp.stack([W_rel2, W_root2, W_rel3, W_root3])
    bb = jnp.stack([b_rel1 + b_root1,
                    b_rel2 + b_root2,
                    b_rel3 + b_root3])

    out = pl.pallas_call(
        _tc_body,
        out_shape=jax.ShapeDtypeStruct((_C, _H), jnp.float32),
    )(ap, p, s, w4, bb)

    return out.reshape(1, _C, _H)


# speedup vs baseline: 1.0066x; 1.0066x over previous
import functools

import jax
import jax.numpy as jnp
from jax import lax
from jax.experimental import pallas as pl
from jax.experimental.pallas import tpu as pltpu
from jax.experimental.pallas import tpu_sc as plsc

_N = 68
_E = 544
_H = 128
_C = 8
_LANES = 16
_NSUB = 4
_NVEC = _E // _LANES
_WVEC = 9


def _sc_build_adj(edge_index):
    mesh = plsc.VectorSubcoreMesh(
        core_axis_name="c", subcore_axis_name="s",
        num_cores=1, num_subcores=_NSUB)

    @functools.partial(
        pl.kernel,
        out_type=jax.ShapeDtypeStruct((_NSUB, _N, _N), jnp.float32),
        mesh=mesh,
        scratch_types=[
            pltpu.VMEM((2, _E), jnp.int32),
            pltpu.VMEM((_N, _N), jnp.float32),
            pltpu.SemaphoreType.DMA,
        ],
        compiler_params=pltpu.CompilerParams(
            needs_layout_passes=False,
            disable_bounds_checks=True,
            disable_semaphore_checks=True,
            skip_device_barrier=True,
        ),
    )
    def build(ei_hbm, out_hbm, ei_v, acc_v, sem):
        sid = lax.axis_index("s")

        @pl.when(sid < _NSUB)
        def _():
            lo = sid * _NVEC // _NSUB
            hi = (sid + 1) * _NVEC // _NSUB
            cp = pltpu.async_copy(ei_hbm, ei_v, sem)

            zeros = jnp.zeros((_LANES,), jnp.float32)

            def zero_body(i, carry):
                for r in range(4):
                    acc_v[i * 4 + r, pl.ds(0, _LANES)] = zeros
                    acc_v[i * 4 + r, pl.ds(16, _LANES)] = zeros
                    acc_v[i * 4 + r, pl.ds(32, _LANES)] = zeros
                    acc_v[i * 4 + r, pl.ds(48, _LANES)] = zeros
                    acc_v[i * 4 + r, pl.ds(_N - _LANES, _LANES)] = zeros
                return carry

            lax.fori_loop(0, _N // 4, zero_body, 0)

            cp.wait()

            ones = jnp.ones((_LANES,), jnp.float32)
            lane = lax.iota(jnp.int32, _LANES)

            def edge_body(j, carry):
                s_ids = ei_v[0, pl.ds((lo + j) * _LANES, _LANES)]
                d_ids = ei_v[1, pl.ds((lo + j) * _LANES, _LANES)]
                for l in range(_LANES):
                    plsc.addupdate_scatter(
                        acc_v, [d_ids, s_ids], ones, mask=lane == l)
                return carry

            lax.fori_loop(0, hi - lo, edge_body, 0)
            pltpu.sync_copy(acc_v, out_hbm.at[sid])

    return build(edge_index)


def _tc_body(ap_ref, x_ref, s_ref, w4_ref, sm_ref, out_ref):
    def dot(p, q):
        return lax.dot_general(p, q, (((1,), (0,)), ((), ())),
                               preferred_element_type=jnp.float32)

    a = jnp.sum(ap_ref[...], axis=0)
    x = x_ref[...]

    c0 = x[:, 0:1]
    c1 = x[:, 1:2]
    xr = c0 * sm_ref[0][None, :] + c1 * sm_ref[1][None, :]
    xo = c0 * sm_ref[2][None, :] + c1 * sm_ref[3][None, :]
    h = jnp.maximum(dot(a, xr) + xo + sm_ref[4][None, :], 0.0)

    for wr, wo, b in ((w4_ref[0], w4_ref[1], sm_ref[5]),
                      (w4_ref[2], w4_ref[3], sm_ref[6])):
        hr = lax.dot_general(h, wr, (((1,), (1,)), ((), ())),
                             preferred_element_type=jnp.float32)
        ho = lax.dot_general(h, wo, (((1,), (1,)), ((), ())),
                             preferred_element_type=jnp.float32)
        h = jnp.maximum(dot(a, hr) + ho + b[None, :], 0.0)

    st = s_ref[0]
    m = jnp.max(st, axis=1, keepdims=True)
    e = jnp.exp(st - m)
    ss = e / jnp.sum(e, axis=1, keepdims=True)
    out_ref[...] = lax.dot_general(ss, h, (((0,), (0,)), ((), ())),
                                   preferred_element_type=jnp.float32)


def kernel(x, edge_index, adj, s,
           W_rel1, b_rel1, W_root1, b_root1,
           W_rel2, b_rel2, W_root2, b_root2,
           W_rel3, b_rel3, W_root3, b_root3):
    del adj
    ap = _sc_build_adj(edge_index)

    w4 = jnp.stack([W_rel2, W_root2, W_rel3, W_root3])
    sm = jnp.concatenate([W_rel1.T, W_root1.T,
                          (b_rel1 + b_root1)[None, :],
                          (b_rel2 + b_root2)[None, :],
                          (b_rel3 + b_root3)[None, :]], axis=0)

    out = pl.pallas_call(
        _tc_body,
        out_shape=jax.ShapeDtypeStruct((_C, _H), jnp.float32),
    )(ap, x, s, w4, sm)

    return out.reshape(1, _C, _H)
